# Initial kernel scaffold; baseline (speedup 1.0000x reference)
import jax
import jax.numpy as jnp
from jax.experimental import pallas as pl


def kernel(x, locations, values):
    raise NotImplementedError("write your pallas kernel here")


# speedup vs baseline: 189.5257x; 189.5257x over previous
import functools

import jax
import jax.numpy as jnp
from jax import lax
from jax.experimental import pallas as pl
from jax.experimental.pallas import tpu as pltpu
from jax.experimental.pallas import tpu_sc as plsc

LANES = 16
NW = 32
CHUNK = 8192


@functools.partial(jax.jit, static_argnames=("n_points", "terms"))
def _sc_pwconst(x, loc_splat, d_splat, n_points, terms):
    s_fns = loc_splat.shape[0]
    per_w = n_points // NW
    n_chunks = per_w // CHUNK
    mesh = plsc.VectorSubcoreMesh(core_axis_name="c", subcore_axis_name="s")

    @functools.partial(
        pl.kernel,
        out_type=jax.ShapeDtypeStruct((s_fns, n_points), jnp.float32),
        mesh=mesh,
        scratch_types=[
            pltpu.VMEM((CHUNK,), jnp.float32),
            pltpu.VMEM((s_fns, CHUNK), jnp.float32),
            pltpu.VMEM((s_fns, terms, LANES), jnp.float32),
            pltpu.VMEM((s_fns, terms, LANES), jnp.float32),
        ],
    )
    def k(x_hbm, loc_hbm, d_hbm, out_hbm, x_v, o_v, loc_v, d_v):
        cid = lax.axis_index("c")
        sid = lax.axis_index("s")
        wid = sid * 2 + cid
        base = wid * per_w
        pltpu.sync_copy(loc_hbm, loc_v)
        pltpu.sync_copy(d_hbm, d_v)

        def chunk_body(ci, carry):
            off = base + ci * CHUNK
            pltpu.sync_copy(x_hbm.at[pl.ds(off, CHUNK)], x_v)
            for s in range(s_fns):
                locs = [loc_v[s, t] for t in range(terms)]
                ds = [d_v[s, t] for t in range(terms)]

                def vec_body(i, c2, locs=locs, ds=ds, s=s):
                    xv = x_v[pl.ds(i * LANES, LANES)]
                    acc = jnp.zeros((LANES,), jnp.float32)
                    for t in range(terms):
                        acc = jnp.where(xv > locs[t], acc + ds[t], acc)
                    o_v[s, pl.ds(i * LANES, LANES)] = acc
                    return c2

                lax.fori_loop(0, CHUNK // LANES, vec_body, 0)
            for s in range(s_fns):
                pltpu.sync_copy(o_v.at[s], out_hbm.at[s, pl.ds(off, CHUNK)])
            return carry

        lax.fori_loop(0, n_chunks, chunk_body, 0)

    return k(x, loc_splat, d_splat)


def kernel(x, locations, values):
    s_fns, L = locations.shape
    n_points = x.shape[0]
    terms = L + 1
    base = values[:, L - 1]
    d0 = values[:, 0] - base
    dl = values[:, 1:] - values[:, :-1]
    d = jnp.concatenate([base[:, None], d0[:, None], dl], axis=1)
    loc = jnp.concatenate(
        [jnp.full((s_fns, 1), -1.0, jnp.float32), locations], axis=1
    )
    loc_splat = jnp.broadcast_to(loc[:, :, None], (s_fns, terms, LANES))
    d_splat = jnp.broadcast_to(d[:, :, None], (s_fns, terms, LANES))
    out = _sc_pwconst(
        x, loc_splat.astype(jnp.float32), d_splat.astype(jnp.float32),
        n_points, terms,
    )
    return out[..., None]


# speedup vs baseline: 492.7640x; 2.6000x over previous
import functools

import jax
import jax.numpy as jnp
from jax import lax
from jax.experimental import pallas as pl
from jax.experimental.pallas import tpu as pltpu
from jax.experimental.pallas import tpu_sc as plsc

LANES = 16
NW = 32
CHUNK = 8192


@functools.partial(jax.jit, static_argnames=("n_points", "terms"))
def _sc_pwconst(x, loc_splat, d_splat, n_points, terms):
    s_fns = loc_splat.shape[0]
    per_w = n_points // NW
    n_chunks = per_w // CHUNK
    mesh = plsc.VectorSubcoreMesh(core_axis_name="c", subcore_axis_name="s")

    @functools.partial(
        pl.kernel,
        out_type=jax.ShapeDtypeStruct((s_fns, n_points), jnp.float32),
        mesh=mesh,
        scratch_types=[
            pltpu.VMEM((CHUNK,), jnp.float32),
            pltpu.VMEM((s_fns, CHUNK), jnp.float32),
            pltpu.VMEM((s_fns, terms, LANES), jnp.float32),
            pltpu.VMEM((s_fns, terms, LANES), jnp.float32),
        ],
    )
    def k(x_hbm, loc_hbm, d_hbm, out_hbm, x_v, o_v, loc_v, d_v):
        cid = lax.axis_index("c")
        sid = lax.axis_index("s")
        wid = sid * 2 + cid
        base = wid * per_w
        pltpu.sync_copy(loc_hbm, loc_v)
        pltpu.sync_copy(d_hbm, d_v)

        def chunk_body(ci, carry):
            off = base + ci * CHUNK
            pltpu.sync_copy(x_hbm.at[pl.ds(off, CHUNK)], x_v)
            for s in range(s_fns):
                locs = [loc_v[s, t] for t in range(terms)]
                ds = [d_v[s, t] for t in range(terms)]

                @plsc.parallel_loop(0, CHUNK // LANES, 1, unroll=8)
                def vec_body(i, locs=locs, ds=ds, s=s):
                    xv = x_v[pl.ds(i * LANES, LANES)]
                    acc = jnp.zeros((LANES,), jnp.float32)
                    for t in range(terms):
                        acc = jnp.where(xv > locs[t], acc + ds[t], acc)
                    o_v[s, pl.ds(i * LANES, LANES)] = acc
            for s in range(s_fns):
                pltpu.sync_copy(o_v.at[s], out_hbm.at[s, pl.ds(off, CHUNK)])
            return carry

        lax.fori_loop(0, n_chunks, chunk_body, 0)

    return k(x, loc_splat, d_splat)


def kernel(x, locations, values):
    s_fns, L = locations.shape
    n_points = x.shape[0]
    terms = L + 1
    base = values[:, L - 1]
    d0 = values[:, 0] - base
    dl = values[:, 1:] - values[:, :-1]
    d = jnp.concatenate([base[:, None], d0[:, None], dl], axis=1)
    loc = jnp.concatenate(
        [jnp.full((s_fns, 1), -1.0, jnp.float32), locations], axis=1
    )
    loc_splat = jnp.broadcast_to(loc[:, :, None], (s_fns, terms, LANES))
    d_splat = jnp.broadcast_to(d[:, :, None], (s_fns, terms, LANES))
    out = _sc_pwconst(
        x, loc_splat.astype(jnp.float32), d_splat.astype(jnp.float32),
        n_points, terms,
    )
    return out[..., None]
